# baseline (speedup 1.0000x reference)
import jax
import jax.numpy as jnp
from jax.experimental import pallas as pl
from jax.experimental.pallas import tpu as pltpu

B = 4096
D = 512
CAP = 8192
TOP_K = 8

BM = 256
BC = 1024
NC = CAP // BC
NB = B // BM


def _row_max(x):
    m = jnp.max(x, axis=0)
    m = jnp.max(m, axis=-1, keepdims=True)
    return m[None]


def _row_sum(x):
    s = jnp.sum(x, axis=0)
    s = jnp.sum(s, axis=-1, keepdims=True)
    return s[None]


def _mem_kernel(cue_ref, p_ref, pb_ref, wgc_ref, wgr_ref, wro_ref, b_ref,
                out_ref, sim_ref, w_ref, acc_ref, z_ref):
    j = pl.program_id(1)

    @pl.when(j < NC)
    def _sim_step():
        cue = cue_ref[...]
        ss = jnp.sum(cue * cue, axis=1, keepdims=True)
        cue_n = cue / jnp.maximum(jnp.sqrt(ss), 1e-12)
        sim_ref[j] = jax.lax.dot_general(
            cue_n, p_ref[...],
            dimension_numbers=(((1,), (1,)), ((), ())),
            preferred_element_type=jnp.float32)

    @pl.when(j == NC - 1)
    def _weights_step():
        sim = sim_ref[...]
        neg = jnp.float32(-jnp.inf)
        m = _row_max(sim)
        for _ in range(TOP_K - 1):
            m = _row_max(jnp.where(sim >= m, neg, sim))
        w = jnp.exp(sim) * (sim >= m).astype(jnp.float32)
        z_ref[...] = _row_sum(w)[0]
        w_ref[...] = w.astype(jnp.bfloat16)

    @pl.when(j >= NC)
    def _combine_step():
        w = w_ref[j - NC]
        contrib = jnp.dot(w, pb_ref[...], preferred_element_type=jnp.float32)

        @pl.when(j == NC)
        def _init():
            acc_ref[...] = contrib

        @pl.when(j > NC)
        def _accum():
            acc_ref[...] += contrib

    @pl.when(j == 2 * NC - 1)
    def _epilogue():
        cue = cue_ref[...]
        retrieved = acc_ref[...] / z_ref[...]
        gate_lin = (jnp.dot(cue, wgc_ref[...], preferred_element_type=jnp.float32)
                    + jnp.dot(retrieved, wgr_ref[...], preferred_element_type=jnp.float32)
                    + b_ref[...])
        gate = jax.nn.sigmoid(gate_lin)
        out_ref[...] = jnp.dot(jnp.tanh(gate * retrieved), wro_ref[...],
                               preferred_element_type=jnp.float32)


def kernel(cue, pattern_store, W_readout, W_gate, b_gate):
    wgc = W_gate[:, :D].T
    wgr = W_gate[:, D:].T
    wro = W_readout.T
    b = b_gate.reshape(1, D)
    p_bf16 = pattern_store.astype(jnp.bfloat16)

    grid = (NB, 2 * NC)
    return pl.pallas_call(
        _mem_kernel,
        grid=grid,
        in_specs=[
            pl.BlockSpec((BM, D), lambda i, j: (i, 0)),
            pl.BlockSpec((BC, D), lambda i, j: (jax.lax.rem(j, NC), 0)),
            pl.BlockSpec((BC, D), lambda i, j: (jax.lax.rem(j, NC), 0)),
            pl.BlockSpec((D, D), lambda i, j: (0, 0)),
            pl.BlockSpec((D, D), lambda i, j: (0, 0)),
            pl.BlockSpec((D, D), lambda i, j: (0, 0)),
            pl.BlockSpec((1, D), lambda i, j: (0, 0)),
        ],
        out_specs=pl.BlockSpec((BM, D), lambda i, j: (i, 0)),
        out_shape=jax.ShapeDtypeStruct((B, D), jnp.float32),
        scratch_shapes=[
            pltpu.VMEM((NC, BM, BC), jnp.float32),
            pltpu.VMEM((NC, BM, BC), jnp.bfloat16),
            pltpu.VMEM((BM, D), jnp.float32),
            pltpu.VMEM((BM, 1), jnp.float32),
        ],
        compiler_params=pltpu.CompilerParams(
            dimension_semantics=("arbitrary", "arbitrary")),
    )(cue, pattern_store, p_bf16, wgc, wgr, wro, b)


# speedup vs baseline: 1.7880x; 1.7880x over previous
import jax
import jax.numpy as jnp
from jax.experimental import pallas as pl
from jax.experimental.pallas import tpu as pltpu

B = 4096
D = 512
CAP = 8192
TOP_K = 8

BM = 512
BC = 1024
NC = CAP // BC
NB = B // BM


def _row_sum(x):
    s = jnp.sum(x, axis=0)
    return jnp.sum(s, axis=-1, keepdims=True)


def _mem_kernel(cue_ref, p_ref, wgc_ref, wgr_ref, wro_ref, b_ref,
                out_ref, sim_ref, acc_ref, z_ref):
    j = pl.program_id(1)

    @pl.when(j < NC)
    def _sim_step():
        cue = cue_ref[...]
        ss = jnp.sum(cue * cue, axis=1, keepdims=True)
        cue_n = cue / jnp.maximum(jnp.sqrt(ss), 1e-12)
        sim_ref[j] = jax.lax.dot_general(
            cue_n, p_ref[...],
            dimension_numbers=(((1,), (1,)), ((), ())),
            preferred_element_type=jnp.float32)

    @pl.when(j == NC - 1)
    def _weights_step():
        sim = sim_ref[...]
        neg = jnp.float32(-jnp.inf)
        a = jnp.max(sim, axis=0)
        b = jnp.max(jnp.where(sim >= a[None], neg, sim), axis=0)
        c = jnp.max(jnp.where(sim >= b[None], neg, sim), axis=0)
        for k in range(TOP_K):
            m = jnp.max(a, axis=-1, keepdims=True)
            if k < TOP_K - 1:
                mask = a >= m
                a = jnp.where(mask, b, a)
                b = jnp.where(mask, c, b)
                c = jnp.where(mask, neg, c)
        t = m[None]
        w = jnp.exp(sim) * (sim >= t).astype(jnp.float32)
        z_ref[...] = _row_sum(w)
        sim_ref[...] = w

    @pl.when(j >= NC)
    def _combine_step():
        w = sim_ref[j - NC]
        contrib = jnp.dot(w, p_ref[...],
                          preferred_element_type=jnp.float32)

        @pl.when(j == NC)
        def _init():
            acc_ref[...] = contrib

        @pl.when(j > NC)
        def _accum():
            acc_ref[...] += contrib

    @pl.when(j == 2 * NC - 1)
    def _epilogue():
        cue = cue_ref[...]
        retrieved = acc_ref[...] / z_ref[...]
        gate_lin = (jnp.dot(cue, wgc_ref[...], preferred_element_type=jnp.float32)
                    + jnp.dot(retrieved, wgr_ref[...], preferred_element_type=jnp.float32)
                    + b_ref[...])
        gate = jax.nn.sigmoid(gate_lin)
        out_ref[...] = jnp.dot(jnp.tanh(gate * retrieved), wro_ref[...],
                               preferred_element_type=jnp.float32)


def kernel(cue, pattern_store, W_readout, W_gate, b_gate):
    wgc = W_gate[:, :D].T
    wgr = W_gate[:, D:].T
    wro = W_readout.T
    b = b_gate.reshape(1, D)

    grid = (NB, 2 * NC)
    return pl.pallas_call(
        _mem_kernel,
        grid=grid,
        in_specs=[
            pl.BlockSpec((BM, D), lambda i, j: (i, 0)),
            pl.BlockSpec((BC, D), lambda i, j: (jax.lax.rem(j, NC), 0)),
            pl.BlockSpec((D, D), lambda i, j: (0, 0)),
            pl.BlockSpec((D, D), lambda i, j: (0, 0)),
            pl.BlockSpec((D, D), lambda i, j: (0, 0)),
            pl.BlockSpec((1, D), lambda i, j: (0, 0)),
        ],
        out_specs=pl.BlockSpec((BM, D), lambda i, j: (i, 0)),
        out_shape=jax.ShapeDtypeStruct((B, D), jnp.float32),
        scratch_shapes=[
            pltpu.VMEM((NC, BM, BC), jnp.float32),
            pltpu.VMEM((BM, D), jnp.float32),
            pltpu.VMEM((BM, 1), jnp.float32),
        ],
        compiler_params=pltpu.CompilerParams(
            dimension_semantics=("arbitrary", "arbitrary")),
    )(cue, pattern_store, wgc, wgr, wro, b)
